# baseline (speedup 1.0000x reference)
import functools

import jax
import jax.numpy as jnp
from jax import lax
from jax.experimental import pallas as pl
from jax.experimental.pallas import tpu as pltpu
from jax.experimental.pallas import tpu_sc as plsc

_C = 85
_S = 256
_NRV = _S // 16
_NCHUNK = 4096 // _S
_PAIRS = 96
_ITEMS = _PAIRS * _NCHUNK
_NW = 32
_ITEMS_PER_W = _ITEMS // _NW
_PAIRS_PER_W = _PAIRS // _NW
_STRIDE = 8.0


def _sc_body(x_hbm, out_hbm, in_v, out_v):
    wid = lax.axis_index("s") * 2 + lax.axis_index("c")
    lane = lax.iota(jnp.int32, 16)
    lane85 = lane * _C

    def pair_body(pi, carry):
        pair = wid * _PAIRS_PER_W + pi
        a = lax.rem(pair, 3)
        aw = jnp.where(a == 0, 10.0, jnp.where(a == 1, 16.0, 33.0))
        ah = jnp.where(a == 0, 13.0, jnp.where(a == 1, 30.0, 23.0))

        def chunk_body(ci, carry2):
            s0 = ci * _S
            pltpu.sync_copy(x_hbm.at[pair, :, pl.ds(s0, _S)], in_v)

            def ch_body(c, carry3):
                for rv in range(_NRV):
                    v = in_v[c, pl.ds(rv * 16, 16)]
                    e = jnp.exp(v)
                    sig = e / (1.0 + e)
                    idx = lane85 + (rv * 16 * _C + c)
                    plsc.store_scatter(out_v, [idx], sig)
                return carry3

            lax.fori_loop(4, _C, ch_body, 0, unroll=False)

            for rv in range(_NRV):
                r_global = lane + (s0 + rv * 16)
                gx = (r_global & 63).astype(jnp.float32)
                gy = (r_global >> 6).astype(jnp.float32)
                base_idx = lane85 + rv * 16 * _C

                v0 = in_v[0, pl.ds(rv * 16, 16)]
                e0 = jnp.exp(v0)
                plsc.store_scatter(out_v, [base_idx],
                                   (e0 / (1.0 + e0) + gx) * _STRIDE)
                v1 = in_v[1, pl.ds(rv * 16, 16)]
                e1 = jnp.exp(v1)
                plsc.store_scatter(out_v, [base_idx + 1],
                                   (e1 / (1.0 + e1) + gy) * _STRIDE)
                v2 = in_v[2, pl.ds(rv * 16, 16)]
                plsc.store_scatter(out_v, [base_idx + 2], jnp.exp(v2) * aw)
                v3 = in_v[3, pl.ds(rv * 16, 16)]
                plsc.store_scatter(out_v, [base_idx + 3], jnp.exp(v3) * ah)

            pltpu.sync_copy(out_v, out_hbm.at[pair * _NCHUNK + ci])
            return carry2

        lax.fori_loop(0, _NCHUNK, chunk_body, 0, unroll=False)
        return carry

    lax.fori_loop(0, _PAIRS_PER_W, pair_body, 0, unroll=False)


_sc_call = functools.partial(
    pl.kernel,
    mesh=plsc.VectorSubcoreMesh(core_axis_name="c", subcore_axis_name="s"),
    out_type=jax.ShapeDtypeStruct((_ITEMS, _S * _C), jnp.float32),
    scratch_types=[
        pltpu.VMEM((_C, _S), jnp.float32),
        pltpu.VMEM((_S * _C,), jnp.float32),
    ],
    compiler_params=pltpu.CompilerParams(needs_layout_passes=False),
)(_sc_body)


@jax.jit
def kernel(x):
    B = x.shape[0]
    x3 = x.reshape(B * 3, _C, 64 * 64)
    out = _sc_call(x3)
    return out.reshape(B, 3 * 64 * 64, _C)
